# Initial kernel scaffold; baseline (speedup 1.0000x reference)
import jax
import jax.numpy as jnp
from jax.experimental import pallas as pl


def kernel(cfeats, regcls, wanchors, hanchors, xanchors, yanchors, W1, b1, W2, b2, W3, b3, W4, b4):
    raise NotImplementedError("write your pallas kernel here")


# speedup vs baseline: 16.8829x; 16.8829x over previous
import jax
import jax.numpy as jnp
from jax.experimental import pallas as pl

_NA = 4
_GX = 6
_GY = 6
_NR = _NA * _GX * _GY
_NC = 17
_POOL = 7
_P2 = _POOL * _POOL
_QK = 16
_KCOLS = _P2 * 200
_KPAD = 7 * 1408


def _qbuild(cft_ref, w1_ref, q_ref):
    cft = cft_ref[...]
    r00 = cft[0:1, :]
    r01 = cft[1:2, :]
    r10 = cft[2:3, :]
    r11 = cft[3:4, :]
    m01 = jnp.maximum(r00, r01)
    m23 = jnp.maximum(r10, r11)
    mx0 = jnp.maximum(r00, r10)
    mx1 = jnp.maximum(r01, r11)
    mall = jnp.maximum(m01, m23)
    zer = jnp.zeros((_QK - 9, 512), jnp.float32)
    vt = jnp.concatenate(
        [r00, r01, m01, r10, r11, m23, mx0, mx1, mall, zer], axis=0)
    q_ref[...] = jnp.dot(vt, w1_ref[...], preferred_element_type=jnp.float32)


def _head(rc_ref, wa_ref, ha_ref, xa_ref, ya_ref, q_ref,
          b1_ref, w2_ref, b2_ref, w3_ref, b3_ref, w4_ref, b4_ref,
          z1_ref, z2_ref):
    rc = rc_ref[...]
    xr = rc[:, 0:1]
    yr = rc[:, 1:2]
    wr = rc[:, 2:3]
    hr = rc[:, 3:4]
    cr = rc[:, 4:5]
    wa = wa_ref[...]
    ha = ha_ref[...]
    xa = xa_ref[...]
    ya = ya_ref[...]

    wreg = jnp.exp(wr) * wa
    hreg = jnp.exp(hr) * ha
    xreg = xr * wa + xa
    yreg = yr * ha + ya
    xis = xreg - wreg * 0.5
    yis = yreg - hreg * 0.5
    xfs = xreg + wreg * 0.5
    yfs = yreg + hreg * 0.5
    c0 = jnp.floor(jnp.maximum(xis, 0.0))
    c1 = jnp.floor(jnp.maximum(yis, 0.0))
    c2 = jnp.minimum(jnp.ceil(xfs), 7.0)
    c3 = jnp.maximum(jnp.ceil(yfs), 7.0)
    c4 = jnp.where((xis < 8.0) & (yis < 8.0) & (xfs >= 0.0) & (yfs >= 0.0),
                   jax.nn.sigmoid(cr), 0.0)
    c0 = jnp.maximum(c0 - 1.0, 0.0)
    c1 = jnp.maximum(c1 - 1.0, 0.0)
    c2 = jnp.minimum(c2, 5.0)
    c3 = jnp.minimum(c3, 5.0)
    c0 = jnp.maximum(c0 - 1.0, 0.0)
    c1 = jnp.maximum(c1 - 1.0, 0.0)
    c2 = jnp.minimum(c2, 3.0)
    c3 = jnp.minimum(c3, 3.0)
    c0 = jnp.floor(c0 * 0.5)
    c1 = jnp.floor(c1 * 0.5)
    c2 = jnp.floor(c2 * 0.5)
    c3 = jnp.floor(c3 * 0.5)

    j = jax.lax.broadcasted_iota(jnp.int32, (_NR, _QK * _P2), 1)
    qidx = (j // _P2).astype(jnp.float32)
    p = j % _P2
    pxf = (p // _POOL).astype(jnp.float32)
    pyf = (p % _POOL).astype(jnp.float32)
    dx = c2 + 1.0 - c0
    dy = c3 + 1.0 - c1
    xi_p = c0 + jnp.floor(pxf * dx * (1.0 / 7.0))
    xf_p = c0 + jnp.floor((pxf + 1.0) * dx * (1.0 / 7.0))
    yi_p = c1 + jnp.floor(pyf * dy * (1.0 / 7.0))
    yf_p = c1 + jnp.floor((pyf + 1.0) * dy * (1.0 / 7.0))
    bx0 = (xi_p <= 0.0) & (xf_p >= 1.0)
    bx1 = (xi_p <= 1.0) & (xf_p >= 2.0)
    by0 = (yi_p <= 0.0) & (yf_p >= 1.0)
    by1 = (yi_p <= 1.0) & (yf_p >= 2.0)
    kx = jnp.where(bx0, 1.0, 0.0) + jnp.where(bx1, 2.0, 0.0)
    ky = jnp.where(by0, 1.0, 0.0) + jnp.where(by1, 2.0, 0.0)
    active = (c4 > 0.0) & (kx > 0.0) & (ky > 0.0)
    codeq = (kx - 1.0) * 3.0 + (ky - 1.0)
    e = jnp.where(active & (codeq == qidx), 1.0, 0.0)

    zpre = jnp.dot(e, q_ref[...], preferred_element_type=jnp.float32)
    h1 = jnp.maximum(zpre + b1_ref[...], 0.0)
    h2 = jnp.maximum(
        jnp.dot(h1, w2_ref[...], preferred_element_type=jnp.float32)
        + b2_ref[...], 0.0)
    z1_ref[...] = (jnp.dot(h2, w3_ref[...], preferred_element_type=jnp.float32)
                   + b3_ref[...])
    z2_ref[...] = (jnp.dot(h2, w4_ref[...], preferred_element_type=jnp.float32)
                   + b4_ref[...])


def kernel(cfeats, regcls, wanchors, hanchors, xanchors, yanchors,
           W1, b1, W2, b2, W3, b3, W4, b4):
    cft = cfeats.reshape(512, 4).T
    w1x = W1.reshape(200, 512, _P2).transpose(1, 2, 0).reshape(512, _KCOLS)
    w1x = jnp.pad(w1x, ((0, 0), (0, _KPAD - _KCOLS)))
    rc = regcls.reshape(5, _NR).T
    wa = wanchors.reshape(_NR, 1)
    ha = hanchors.reshape(_NR, 1)
    xa = xanchors.reshape(_NR, 1)
    ya = yanchors.reshape(_NR, 1)

    qcols = pl.pallas_call(
        _qbuild,
        grid=(7,),
        in_specs=[
            pl.BlockSpec((4, 512), lambda i: (0, 0)),
            pl.BlockSpec((512, 1408), lambda i: (0, i)),
        ],
        out_specs=pl.BlockSpec((_QK, 1408), lambda i: (0, i)),
        out_shape=jax.ShapeDtypeStruct((_QK, _KPAD), jnp.float32),
    )(cft, w1x)

    qr = qcols[:, :_KCOLS].reshape(_QK * _P2, 200)

    z1f, z2f = pl.pallas_call(
        _head,
        out_shape=(
            jax.ShapeDtypeStruct((_NR, _NC), jnp.float32),
            jax.ShapeDtypeStruct((_NR, 4 * _NC), jnp.float32),
        ),
    )(rc, wa, ha, xa, ya, qr,
      b1.reshape(1, 200), W2.T, b2.reshape(1, 100),
      W3.T, b3.reshape(1, _NC), W4.T, b4.reshape(1, 4 * _NC))

    z1 = z1f.reshape(1, _NA, _GX, _GY, _NC)
    z3 = z2f.reshape(_NA, _GX, _GY, 4, _NC).transpose(3, 0, 1, 2, 4)[None]
    return (z1, z3)


# speedup vs baseline: 34.2318x; 2.0276x over previous
import jax
import jax.numpy as jnp
from jax.experimental import pallas as pl

_NA = 4
_GX = 6
_GY = 6
_NR = _NA * _GX * _GY
_NC = 17
_POOL = 7
_P2 = _POOL * _POOL
_QK = 16


def _vtable(cf):
    f00 = cf[:, 0:1]
    f01 = cf[:, 1:2]
    f10 = cf[:, 2:3]
    f11 = cf[:, 3:4]
    m01 = jnp.maximum(f00, f01)
    m23 = jnp.maximum(f10, f11)
    mx0 = jnp.maximum(f00, f10)
    mx1 = jnp.maximum(f01, f11)
    mall = jnp.maximum(m01, m23)
    zer = jnp.zeros((512, _QK - 9), jnp.float32)
    return jnp.concatenate(
        [f00, f01, m01, f10, f11, m23, mx0, mx1, mall, zer], axis=1)


def _qbuild(w1_ref, cf_ref, q_ref):
    v = _vtable(cf_ref[...])
    x = w1_ref[...]
    y = x.reshape(8, 512, _P2)
    z = jnp.swapaxes(y, 1, 2)
    w = z.reshape(8 * _P2, 512)
    q_ref[...] = jnp.dot(w, v, preferred_element_type=jnp.float32)


def _head(rc_ref, wa_ref, ha_ref, xa_ref, ya_ref, q_ref,
          b1_ref, w2_ref, b2_ref, w3_ref, b3_ref, w4_ref, b4_ref,
          z1_ref, z2_ref):
    rc = rc_ref[...]
    xr = rc[:, 0:1]
    yr = rc[:, 1:2]
    wr = rc[:, 2:3]
    hr = rc[:, 3:4]
    cr = rc[:, 4:5]
    wa = wa_ref[...]
    ha = ha_ref[...]
    xa = xa_ref[...]
    ya = ya_ref[...]

    wreg = jnp.exp(wr) * wa
    hreg = jnp.exp(hr) * ha
    xreg = xr * wa + xa
    yreg = yr * ha + ya
    xis = xreg - wreg * 0.5
    yis = yreg - hreg * 0.5
    xfs = xreg + wreg * 0.5
    yfs = yreg + hreg * 0.5
    c0 = jnp.floor(jnp.maximum(xis, 0.0))
    c1 = jnp.floor(jnp.maximum(yis, 0.0))
    c2 = jnp.minimum(jnp.ceil(xfs), 7.0)
    c3 = jnp.maximum(jnp.ceil(yfs), 7.0)
    c4 = jnp.where((xis < 8.0) & (yis < 8.0) & (xfs >= 0.0) & (yfs >= 0.0),
                   jax.nn.sigmoid(cr), 0.0)
    c0 = jnp.maximum(c0 - 1.0, 0.0)
    c1 = jnp.maximum(c1 - 1.0, 0.0)
    c2 = jnp.minimum(c2, 5.0)
    c3 = jnp.minimum(c3, 5.0)
    c0 = jnp.maximum(c0 - 1.0, 0.0)
    c1 = jnp.maximum(c1 - 1.0, 0.0)
    c2 = jnp.minimum(c2, 3.0)
    c3 = jnp.minimum(c3, 3.0)
    c0 = jnp.floor(c0 * 0.5)
    c1 = jnp.floor(c1 * 0.5)
    c2 = jnp.floor(c2 * 0.5)
    c3 = jnp.floor(c3 * 0.5)

    j = jax.lax.broadcasted_iota(jnp.int32, (_NR, _QK * _P2), 1)
    qidx = (j % _QK).astype(jnp.float32)
    p = j // _QK
    pxf = (p // _POOL).astype(jnp.float32)
    pyf = (p % _POOL).astype(jnp.float32)
    dx = c2 + 1.0 - c0
    dy = c3 + 1.0 - c1
    xi_p = c0 + jnp.floor(pxf * dx * (1.0 / 7.0))
    xf_p = c0 + jnp.floor((pxf + 1.0) * dx * (1.0 / 7.0))
    yi_p = c1 + jnp.floor(pyf * dy * (1.0 / 7.0))
    yf_p = c1 + jnp.floor((pyf + 1.0) * dy * (1.0 / 7.0))
    bx0 = (xi_p <= 0.0) & (xf_p >= 1.0)
    bx1 = (xi_p <= 1.0) & (xf_p >= 2.0)
    by0 = (yi_p <= 0.0) & (yf_p >= 1.0)
    by1 = (yi_p <= 1.0) & (yf_p >= 2.0)
    kx = jnp.where(bx0, 1.0, 0.0) + jnp.where(bx1, 2.0, 0.0)
    ky = jnp.where(by0, 1.0, 0.0) + jnp.where(by1, 2.0, 0.0)
    active = (c4 > 0.0) & (kx > 0.0) & (ky > 0.0)
    codeq = (kx - 1.0) * 3.0 + (ky - 1.0)
    e = jnp.where(active & (codeq == qidx), 1.0, 0.0)

    qt = jnp.swapaxes(q_ref[...], 0, 1)
    zpre = jnp.dot(e, qt, preferred_element_type=jnp.float32)
    h1 = jnp.maximum(zpre + b1_ref[...], 0.0)
    h2 = jnp.maximum(
        jnp.dot(h1, w2_ref[...], preferred_element_type=jnp.float32)
        + b2_ref[...], 0.0)
    z1_ref[...] = (jnp.dot(h2, w3_ref[...], preferred_element_type=jnp.float32)
                   + b3_ref[...])
    z2_ref[...] = (jnp.dot(h2, w4_ref[...], preferred_element_type=jnp.float32)
                   + b4_ref[...])


def kernel(cfeats, regcls, wanchors, hanchors, xanchors, yanchors,
           W1, b1, W2, b2, W3, b3, W4, b4):
    cf = cfeats.reshape(512, 4)
    rc = regcls.reshape(5, _NR).T
    wa = wanchors.reshape(_NR, 1)
    ha = hanchors.reshape(_NR, 1)
    xa = xanchors.reshape(_NR, 1)
    ya = yanchors.reshape(_NR, 1)

    q3 = pl.pallas_call(
        _qbuild,
        grid=(25,),
        in_specs=[
            pl.BlockSpec((8, 512 * _P2), lambda i: (i, 0)),
            pl.BlockSpec((512, 4), lambda i: (0, 0)),
        ],
        out_specs=pl.BlockSpec((8 * _P2, _QK), lambda i: (i, 0)),
        out_shape=jax.ShapeDtypeStruct((200 * _P2, _QK), jnp.float32),
    )(W1, cf)

    qr = q3.reshape(200, _P2 * _QK)

    z1f, z2f = pl.pallas_call(
        _head,
        out_shape=(
            jax.ShapeDtypeStruct((_NR, _NC), jnp.float32),
            jax.ShapeDtypeStruct((_NR, 4 * _NC), jnp.float32),
        ),
    )(rc, wa, ha, xa, ya, qr,
      b1.reshape(1, 200), W2.T, b2.reshape(1, 100),
      W3.T, b3.reshape(1, _NC), W4.T, b4.reshape(1, 4 * _NC))

    z1 = z1f.reshape(1, _NA, _GX, _GY, _NC)
    z3 = z2f.reshape(_NA, _GX, _GY, 4, _NC).transpose(3, 0, 1, 2, 4)[None]
    return (z1, z3)


# speedup vs baseline: 51.8118x; 1.5136x over previous
import jax
import jax.numpy as jnp
from jax.experimental import pallas as pl

_NA = 4
_GX = 6
_GY = 6
_NR = _NA * _GX * _GY
_NC = 17
_POOL = 7
_P2 = _POOL * _POOL
_QK = 16


def _vtable(cf):
    f00 = cf[:, 0:1]
    f01 = cf[:, 1:2]
    f10 = cf[:, 2:3]
    f11 = cf[:, 3:4]
    m01 = jnp.maximum(f00, f01)
    m23 = jnp.maximum(f10, f11)
    mx0 = jnp.maximum(f00, f10)
    mx1 = jnp.maximum(f01, f11)
    mall = jnp.maximum(m01, m23)
    zer = jnp.zeros((512, _QK - 9), jnp.float32)
    return jnp.concatenate(
        [f00, f01, m01, f10, f11, m23, mx0, mx1, mall, zer], axis=1)


def _qbuild(w1_ref, cf_ref, q_ref):
    v = _vtable(cf_ref[...]).astype(jnp.bfloat16)
    x = w1_ref[...].astype(jnp.bfloat16)
    y = x.reshape(40, 512, _P2)
    z = jnp.swapaxes(y, 1, 2)
    w = z.reshape(40 * _P2, 512)
    q_ref[...] = jnp.dot(w, v, preferred_element_type=jnp.float32)


def _head(rc_ref, wa_ref, ha_ref, xa_ref, ya_ref, q_ref,
          b1_ref, w2_ref, b2_ref, w3_ref, b3_ref, w4_ref, b4_ref,
          z1_ref, z2_ref):
    rc = rc_ref[...]
    xr = rc[:, 0:1]
    yr = rc[:, 1:2]
    wr = rc[:, 2:3]
    hr = rc[:, 3:4]
    cr = rc[:, 4:5]
    wa = wa_ref[...]
    ha = ha_ref[...]
    xa = xa_ref[...]
    ya = ya_ref[...]

    wreg = jnp.exp(wr) * wa
    hreg = jnp.exp(hr) * ha
    xreg = xr * wa + xa
    yreg = yr * ha + ya
    xis = xreg - wreg * 0.5
    yis = yreg - hreg * 0.5
    xfs = xreg + wreg * 0.5
    yfs = yreg + hreg * 0.5
    c0 = jnp.floor(jnp.maximum(xis, 0.0))
    c1 = jnp.floor(jnp.maximum(yis, 0.0))
    c2 = jnp.minimum(jnp.ceil(xfs), 7.0)
    c3 = jnp.maximum(jnp.ceil(yfs), 7.0)
    c4 = jnp.where((xis < 8.0) & (yis < 8.0) & (xfs >= 0.0) & (yfs >= 0.0),
                   jax.nn.sigmoid(cr), 0.0)
    c0 = jnp.maximum(c0 - 1.0, 0.0)
    c1 = jnp.maximum(c1 - 1.0, 0.0)
    c2 = jnp.minimum(c2, 5.0)
    c3 = jnp.minimum(c3, 5.0)
    c0 = jnp.maximum(c0 - 1.0, 0.0)
    c1 = jnp.maximum(c1 - 1.0, 0.0)
    c2 = jnp.minimum(c2, 3.0)
    c3 = jnp.minimum(c3, 3.0)
    c0 = jnp.floor(c0 * 0.5)
    c1 = jnp.floor(c1 * 0.5)
    c2 = jnp.floor(c2 * 0.5)
    c3 = jnp.floor(c3 * 0.5)

    j = jax.lax.broadcasted_iota(jnp.int32, (_NR, _QK * _P2), 1)
    qidx = (j % _QK).astype(jnp.float32)
    p = j // _QK
    pxf = (p // _POOL).astype(jnp.float32)
    pyf = (p % _POOL).astype(jnp.float32)
    dx = c2 + 1.0 - c0
    dy = c3 + 1.0 - c1
    xi_p = c0 + jnp.floor(pxf * dx * (1.0 / 7.0))
    xf_p = c0 + jnp.floor((pxf + 1.0) * dx * (1.0 / 7.0))
    yi_p = c1 + jnp.floor(pyf * dy * (1.0 / 7.0))
    yf_p = c1 + jnp.floor((pyf + 1.0) * dy * (1.0 / 7.0))
    bx0 = (xi_p <= 0.0) & (xf_p >= 1.0)
    bx1 = (xi_p <= 1.0) & (xf_p >= 2.0)
    by0 = (yi_p <= 0.0) & (yf_p >= 1.0)
    by1 = (yi_p <= 1.0) & (yf_p >= 2.0)
    kx = jnp.where(bx0, 1.0, 0.0) + jnp.where(bx1, 2.0, 0.0)
    ky = jnp.where(by0, 1.0, 0.0) + jnp.where(by1, 2.0, 0.0)
    active = (c4 > 0.0) & (kx > 0.0) & (ky > 0.0)
    codeq = (kx - 1.0) * 3.0 + (ky - 1.0)
    e = jnp.where(active & (codeq == qidx), 1.0, 0.0)

    qt = jnp.swapaxes(q_ref[...], 0, 1)
    zpre = jnp.dot(e, qt, preferred_element_type=jnp.float32)
    h1 = jnp.maximum(zpre + b1_ref[...], 0.0)
    h2 = jnp.maximum(
        jnp.dot(h1, w2_ref[...], preferred_element_type=jnp.float32)
        + b2_ref[...], 0.0)
    z1_ref[...] = (jnp.dot(h2, w3_ref[...], preferred_element_type=jnp.float32)
                   + b3_ref[...])
    z2_ref[...] = (jnp.dot(h2, w4_ref[...], preferred_element_type=jnp.float32)
                   + b4_ref[...])


def kernel(cfeats, regcls, wanchors, hanchors, xanchors, yanchors,
           W1, b1, W2, b2, W3, b3, W4, b4):
    cf = cfeats.reshape(512, 4)
    rc = regcls.reshape(5, _NR).T
    wa = wanchors.reshape(_NR, 1)
    ha = hanchors.reshape(_NR, 1)
    xa = xanchors.reshape(_NR, 1)
    ya = yanchors.reshape(_NR, 1)

    q3 = pl.pallas_call(
        _qbuild,
        grid=(5,),
        in_specs=[
            pl.BlockSpec((40, 512 * _P2), lambda i: (i, 0)),
            pl.BlockSpec((512, 4), lambda i: (0, 0)),
        ],
        out_specs=pl.BlockSpec((40 * _P2, _QK), lambda i: (i, 0)),
        out_shape=jax.ShapeDtypeStruct((200 * _P2, _QK), jnp.float32),
    )(W1, cf)

    qr = q3.reshape(200, _P2 * _QK)

    z1f, z2f = pl.pallas_call(
        _head,
        out_shape=(
            jax.ShapeDtypeStruct((_NR, _NC), jnp.float32),
            jax.ShapeDtypeStruct((_NR, 4 * _NC), jnp.float32),
        ),
    )(rc, wa, ha, xa, ya, qr,
      b1.reshape(1, 200), W2.T, b2.reshape(1, 100),
      W3.T, b3.reshape(1, _NC), W4.T, b4.reshape(1, 4 * _NC))

    z1 = z1f.reshape(1, _NA, _GX, _GY, _NC)
    z3 = z2f.reshape(_NA, _GX, _GY, 4, _NC).transpose(3, 0, 1, 2, 4)[None]
    return (z1, z3)


# speedup vs baseline: 52.7231x; 1.0176x over previous
import jax
import jax.numpy as jnp
from jax.experimental import pallas as pl

_NA = 4
_GX = 6
_GY = 6
_NR = _NA * _GX * _GY
_NC = 17
_POOL = 7
_P2 = _POOL * _POOL
_QK = 16


def _vtable(cf):
    f00 = cf[:, 0:1]
    f01 = cf[:, 1:2]
    f10 = cf[:, 2:3]
    f11 = cf[:, 3:4]
    m01 = jnp.maximum(f00, f01)
    m23 = jnp.maximum(f10, f11)
    mx0 = jnp.maximum(f00, f10)
    mx1 = jnp.maximum(f01, f11)
    mall = jnp.maximum(m01, m23)
    zer = jnp.zeros((512, _QK - 9), jnp.float32)
    return jnp.concatenate(
        [f00, f01, m01, f10, f11, m23, mx0, mx1, mall, zer], axis=1)


def _qbuild(w1_ref, cf_ref, q_ref):
    v = _vtable(cf_ref[...]).astype(jnp.bfloat16)
    x = w1_ref[...].astype(jnp.bfloat16)
    y = x.reshape(40, 512, _P2)
    z = jnp.swapaxes(y, 1, 2)
    w = z.reshape(40 * _P2, 512)
    q_ref[...] = jnp.dot(w, v, preferred_element_type=jnp.float32)


def _head(rc_ref, wa_ref, ha_ref, xa_ref, ya_ref, q_ref,
          b1_ref, w2_ref, b2_ref, w3_ref, b3_ref, w4_ref, b4_ref,
          z1_ref, z2_ref):
    rc = jnp.swapaxes(rc_ref[...], 0, 1)
    xr = rc[:, 0:1]
    yr = rc[:, 1:2]
    wr = rc[:, 2:3]
    hr = rc[:, 3:4]
    cr = rc[:, 4:5]
    wa = wa_ref[...]
    ha = ha_ref[...]
    xa = xa_ref[...]
    ya = ya_ref[...]

    wreg = jnp.exp(wr) * wa
    hreg = jnp.exp(hr) * ha
    xreg = xr * wa + xa
    yreg = yr * ha + ya
    xis = xreg - wreg * 0.5
    yis = yreg - hreg * 0.5
    xfs = xreg + wreg * 0.5
    yfs = yreg + hreg * 0.5
    c0 = jnp.floor(jnp.maximum(xis, 0.0))
    c1 = jnp.floor(jnp.maximum(yis, 0.0))
    c2 = jnp.minimum(jnp.ceil(xfs), 7.0)
    c3 = jnp.maximum(jnp.ceil(yfs), 7.0)
    c4 = jnp.where((xis < 8.0) & (yis < 8.0) & (xfs >= 0.0) & (yfs >= 0.0),
                   jax.nn.sigmoid(cr), 0.0)
    c0 = jnp.maximum(c0 - 1.0, 0.0)
    c1 = jnp.maximum(c1 - 1.0, 0.0)
    c2 = jnp.minimum(c2, 5.0)
    c3 = jnp.minimum(c3, 5.0)
    c0 = jnp.maximum(c0 - 1.0, 0.0)
    c1 = jnp.maximum(c1 - 1.0, 0.0)
    c2 = jnp.minimum(c2, 3.0)
    c3 = jnp.minimum(c3, 3.0)
    c0 = jnp.floor(c0 * 0.5)
    c1 = jnp.floor(c1 * 0.5)
    c2 = jnp.floor(c2 * 0.5)
    c3 = jnp.floor(c3 * 0.5)

    j = jax.lax.broadcasted_iota(jnp.int32, (_NR, _QK * _P2), 1)
    qidx = (j % _QK).astype(jnp.float32)
    p = j // _QK
    pxf = (p // _POOL).astype(jnp.float32)
    pyf = (p % _POOL).astype(jnp.float32)
    dx = c2 + 1.0 - c0
    dy = c3 + 1.0 - c1
    xi_p = c0 + jnp.floor(pxf * dx * (1.0 / 7.0))
    xf_p = c0 + jnp.floor((pxf + 1.0) * dx * (1.0 / 7.0))
    yi_p = c1 + jnp.floor(pyf * dy * (1.0 / 7.0))
    yf_p = c1 + jnp.floor((pyf + 1.0) * dy * (1.0 / 7.0))
    bx0 = (xi_p <= 0.0) & (xf_p >= 1.0)
    bx1 = (xi_p <= 1.0) & (xf_p >= 2.0)
    by0 = (yi_p <= 0.0) & (yf_p >= 1.0)
    by1 = (yi_p <= 1.0) & (yf_p >= 2.0)
    kx = jnp.where(bx0, 1.0, 0.0) + jnp.where(bx1, 2.0, 0.0)
    ky = jnp.where(by0, 1.0, 0.0) + jnp.where(by1, 2.0, 0.0)
    active = (c4 > 0.0) & (kx > 0.0) & (ky > 0.0)
    codeq = (kx - 1.0) * 3.0 + (ky - 1.0)
    e = jnp.where(active & (codeq == qidx), 1.0, 0.0)

    qt = jnp.swapaxes(q_ref[...], 0, 1)
    zpre = jnp.dot(e, qt, preferred_element_type=jnp.float32)
    h1 = jnp.maximum(zpre + b1_ref[...], 0.0)
    w2t = jnp.swapaxes(w2_ref[...], 0, 1)
    h2 = jnp.maximum(
        jnp.dot(h1, w2t, preferred_element_type=jnp.float32)
        + b2_ref[...], 0.0)
    w3t = jnp.swapaxes(w3_ref[...], 0, 1)
    w4t = jnp.swapaxes(w4_ref[...], 0, 1)
    z1_ref[...] = (jnp.dot(h2, w3t, preferred_element_type=jnp.float32)
                   + b3_ref[...])
    zz = (jnp.dot(h2, w4t, preferred_element_type=jnp.float32)
          + b4_ref[...])
    for s in range(4):
        z2_ref[pl.ds(s * _NR, _NR), :] = zz[:, s * _NC:(s + 1) * _NC]


def kernel(cfeats, regcls, wanchors, hanchors, xanchors, yanchors,
           W1, b1, W2, b2, W3, b3, W4, b4):
    cf = cfeats.reshape(512, 4)
    rc = regcls.reshape(5, _NR)
    wa = wanchors.reshape(_NR, 1)
    ha = hanchors.reshape(_NR, 1)
    xa = xanchors.reshape(_NR, 1)
    ya = yanchors.reshape(_NR, 1)

    q3 = pl.pallas_call(
        _qbuild,
        grid=(5,),
        in_specs=[
            pl.BlockSpec((40, 512 * _P2), lambda i: (i, 0)),
            pl.BlockSpec((512, 4), lambda i: (0, 0)),
        ],
        out_specs=pl.BlockSpec((40 * _P2, _QK), lambda i: (i, 0)),
        out_shape=jax.ShapeDtypeStruct((200 * _P2, _QK), jnp.float32),
    )(W1, cf)

    qr = q3.reshape(200, _P2 * _QK)

    z1f, z2f = pl.pallas_call(
        _head,
        out_shape=(
            jax.ShapeDtypeStruct((_NR, _NC), jnp.float32),
            jax.ShapeDtypeStruct((4 * _NR, _NC), jnp.float32),
        ),
    )(rc, wa, ha, xa, ya, qr,
      b1.reshape(1, 200), W2, b2.reshape(1, 100),
      W3, b3.reshape(1, _NC), W4, b4.reshape(1, 4 * _NC))

    z1 = z1f.reshape(1, _NA, _GX, _GY, _NC)
    z3 = z2f.reshape(1, 4, _NA, _GX, _GY, _NC)
    return (z1, z3)
